# Initial kernel scaffold; baseline (speedup 1.0000x reference)
import jax
import jax.numpy as jnp
from jax.experimental import pallas as pl


def kernel(x, edge_index, W1, b1, W2, b2):
    raise NotImplementedError("write your pallas kernel here")


# speedup vs baseline: 281.2042x; 281.2042x over previous
import functools

import jax
import jax.numpy as jnp
from jax import lax
from jax.experimental import pallas as pl
from jax.experimental.pallas import tpu as pltpu
from jax.experimental.pallas import tpu_sc as plsc

NC = 2
NS = 16
NW = NC * NS
L = 16
ROW = 128
K = 16


def _make_edge_pass(mode, n_pad, chunks):
  acc_mult = 2 if mode == 2 else 1
  acc_n = n_pad * acc_mult
  slc = n_pad // NS
  rows_per_tile = chunks * K
  mesh = plsc.VectorSubcoreMesh(core_axis_name="c", subcore_axis_name="s")

  half = slc // 2
  scratch = [
      pltpu.VMEM_SHARED((acc_n,), jnp.float32),
      pltpu.VMEM((2, K, ROW), jnp.int32),
      pltpu.VMEM((half,), jnp.float32),
      pltpu.SemaphoreType.DMA,
      pltpu.SemaphoreType.DMA,
      pltpu.SemaphoreType.DMA,
  ]
  if mode == 0:
    scratch.append(pltpu.VMEM((ROW,), jnp.float32))
  else:
    scratch.append(pltpu.VMEM((2, K, ROW), jnp.float32))
    scratch.append(pltpu.VMEM((n_pad,), jnp.float32))
    scratch.append(pltpu.VMEM((2, K, ROW), jnp.int32))

  def body(*refs):
    if mode == 0:
      (dst_hbm, zeros_hbm, out_hbm, acc, dst_v, zbuf, sem0, sem1, sem_sc,
       ones_v) = refs
    else:
      (src_hbm, dst_hbm, tab_hbm, zeros_hbm, out_hbm, acc, dst_v, zbuf, sem0,
       sem1, sem_sc, val_v, tab_v, src_v) = refs
    sems = (sem0, sem1)

    cid = lax.axis_index("c")
    sid = lax.axis_index("s")
    wid = sid * NC + cid
    base_rows = wid * rows_per_tile

    off0 = sid * acc_mult * slc
    pltpu.sync_copy(zeros_hbm.at[pl.ds(0, half)], zbuf)
    for h in range(acc_mult * 2):
      pltpu.sync_copy(zbuf, acc.at[pl.ds(off0 + h * half, half)])

    if mode == 0:
      one16 = jnp.ones((L,), jnp.float32)

      @pl.loop(0, ROW // L)
      def _ones(i):
        ones_v[pl.ds(i * L, L)] = one16
    else:
      pltpu.sync_copy(tab_hbm, tab_v)

    plsc.subcore_barrier()

    def load_chunk(c, b):
      r = base_rows + c * K
      pltpu.async_copy(dst_hbm.at[pl.ds(r, K)], dst_v.at[b], sems[b])
      if mode != 0:
        pltpu.async_copy(src_hbm.at[pl.ds(r, K)], src_v.at[b], sems[b])

    def wait_chunk(c, b):
      r = base_rows + c * K
      pltpu.make_async_copy(dst_hbm.at[pl.ds(r, K)], dst_v.at[b], sems[b]).wait()
      if mode != 0:
        pltpu.make_async_copy(src_hbm.at[pl.ds(r, K)], src_v.at[b],
                              sems[b]).wait()

    load_chunk(0, 0)

    @pl.loop(0, chunks // 2)
    def _main(cc):
      for b in (0, 1):
        c = cc * 2 + b
        if b == 0:
          load_chunk(c + 1, 1)
        else:
          @pl.when(cc < chunks // 2 - 1)
          def _pf():
            load_chunk(c + 1, 0)
        wait_chunk(c, b)

        if mode != 0:
          for j in range(K):
            @pl.loop(0, ROW // L)
            def _gather(g, j=j, b=b):
              col = g * L
              s16 = src_v[b, j, pl.ds(col, L)]
              vals = plsc.load_gather(tab_v, [s16])
              if mode == 2:
                d16 = dst_v[b, j, pl.ds(col, L)]
                route = jnp.where(vals < 0.0, jnp.int32(n_pad), jnp.int32(0))
                dst_v[b, j, pl.ds(col, L)] = d16 + route
              val_v[b, j, pl.ds(col, L)] = vals

        descs = []
        for j in range(K):
          src_ref = ones_v if mode == 0 else val_v.at[b, j]
          descs.append(
              pltpu.async_copy(src_ref, acc.at[dst_v.at[b, j]], sem_sc,
                               add=True))
        for d in descs:
          d.wait()

    plsc.subcore_barrier()

    for h in range(acc_mult * 2):
      pltpu.sync_copy(acc.at[pl.ds(off0 + h * half, half)], zbuf)
      pltpu.sync_copy(zbuf, out_hbm.at[pl.ds(cid * acc_n + off0 + h * half,
                                             half)])

  out_type = jax.ShapeDtypeStruct((NC * acc_n,), jnp.float32)
  return pl.kernel(
      body, out_type=out_type, mesh=mesh, scratch_types=scratch,
      compiler_params=pltpu.CompilerParams(needs_layout_passes=False))


def _tc1_body(degp_ref, x_ref, dis_ref, y_ref):
  deg = degp_ref[0] + degp_ref[1] + 1.0
  dis = lax.rsqrt(deg)
  dis_ref[...] = dis
  y_ref[...] = x_ref[...] * dis


def _tc2_body(t1p_ref, y_ref, dis_ref, sy_ref):
  t1 = t1p_ref[0] + t1p_ref[1]
  dis = dis_ref[...]
  sy_ref[...] = dis * dis * (t1 + y_ref[...])


def _tc3_body(tp_ref, sy_ref, dis_ref, par_ref, o0_ref, o1_ref):
  tu = tp_ref[0, 0] + tp_ref[1, 0]
  tv = tp_ref[0, 1] + tp_ref[1, 1]
  sy = sy_ref[...]
  dis = dis_ref[...]
  a = dis * (tu + jnp.maximum(sy, 0.0))
  c = dis * (tv + jnp.minimum(sy, 0.0))
  l0 = a * par_ref[0] + c * par_ref[2] + par_ref[4]
  l1 = a * par_ref[1] + c * par_ref[3] + par_ref[5]
  m = jnp.maximum(l0, l1)
  lse = m + jnp.log(jnp.exp(l0 - m) + jnp.exp(l1 - m))
  o0_ref[...] = l0 - lse
  o1_ref[...] = l1 - lse


def kernel(x, edge_index, W1, b1, W2, b2):
  n = x.shape[0]
  e = edge_index.shape[1]
  n_pad = ((n + 1 + 127) // 128) * 128
  r = n_pad // 128
  chunk_e = K * ROW
  chunks = -(-e // (NW * chunk_e))
  chunks += chunks % 2
  e_pad = NW * chunks * chunk_e

  src = jnp.concatenate(
      [edge_index[0], jnp.zeros((e_pad - e,), jnp.int32)]).reshape(-1, ROW)
  dst = jnp.concatenate(
      [edge_index[1], jnp.full((e_pad - e,), n, jnp.int32)]).reshape(-1, ROW)
  xp = jnp.pad(x[:, 0], (0, n_pad - n)).reshape(r, 128)

  deg_pass = _make_edge_pass(0, n_pad, chunks)
  sum_pass = _make_edge_pass(1, n_pad, chunks)
  sign_pass = _make_edge_pass(2, n_pad, chunks)

  zeros1 = jnp.zeros((n_pad,), jnp.float32)
  zeros2 = jnp.zeros((2 * n_pad,), jnp.float32)

  degp = deg_pass(dst, zeros1)

  tc1 = pl.pallas_call(
      _tc1_body,
      out_shape=[jax.ShapeDtypeStruct((r, 128), jnp.float32)] * 2,
  )
  dis2, y2 = tc1(degp.reshape(NC, r, 128), xp)

  t1p = sum_pass(src, dst, y2.reshape(-1), zeros1)

  tc2 = pl.pallas_call(
      _tc2_body,
      out_shape=jax.ShapeDtypeStruct((r, 128), jnp.float32),
  )
  sy2 = tc2(t1p.reshape(NC, r, 128), y2, dis2)

  tp = sign_pass(src, dst, sy2.reshape(-1), zeros2)

  w1 = W1[0]
  wp2 = jnp.maximum(w1, 0.0) @ W2
  wm2 = jnp.minimum(w1, 0.0) @ W2
  par = jnp.stack([wp2[0], wp2[1], wm2[0], wm2[1], b2[0], b2[1],
                   jnp.float32(0), jnp.float32(0)])

  tc3 = pl.pallas_call(
      _tc3_body,
      in_specs=[
          pl.BlockSpec(),
          pl.BlockSpec(),
          pl.BlockSpec(),
          pl.BlockSpec(memory_space=pltpu.SMEM),
      ],
      out_shape=[jax.ShapeDtypeStruct((r, 128), jnp.float32)] * 2,
  )
  o0, o1 = tc3(tp.reshape(NC, 2, r, 128), sy2, dis2, par)

  return jnp.stack([o0.reshape(-1)[:n], o1.reshape(-1)[:n]], axis=1)


# speedup vs baseline: 341.5350x; 1.2145x over previous
import functools

import jax
import jax.numpy as jnp
from jax import lax
from jax.experimental import pallas as pl
from jax.experimental.pallas import tpu as pltpu
from jax.experimental.pallas import tpu_sc as plsc

NC = 2
NS = 16
NW = NC * NS
L = 16
ROW = 128
K = 8
NB = 4


def _make_edge_pass(mode, n_pad, chunks):
  acc_mult = 2 if mode == 2 else 1
  acc_n = n_pad * acc_mult
  slc = n_pad // NS
  rows_per_tile = chunks * K
  mesh = plsc.VectorSubcoreMesh(core_axis_name="c", subcore_axis_name="s")

  half = slc // 2
  scratch = [
      pltpu.VMEM_SHARED((acc_n,), jnp.float32),
      pltpu.VMEM((NB, K, ROW), jnp.int32),
      pltpu.VMEM((half,), jnp.float32),
  ]
  scratch += [pltpu.SemaphoreType.DMA] * NB
  scratch += [pltpu.SemaphoreType.DMA] * NB
  if mode == 0:
    scratch.append(pltpu.VMEM((ROW,), jnp.float32))
  else:
    scratch.append(pltpu.VMEM((NB, K, ROW), jnp.float32))
    scratch.append(pltpu.VMEM((n_pad,), jnp.float32))
    scratch.append(pltpu.VMEM((NB, K, ROW), jnp.int32))

  def body(*refs):
    if mode == 0:
      (dst_hbm, zeros_hbm, out_hbm, acc, dst_v, zbuf, *rest) = refs
      ones_v = rest[2 * NB]
    else:
      (src_hbm, dst_hbm, tab_hbm, zeros_hbm, out_hbm, acc, dst_v, zbuf,
       *rest) = refs
      val_v, tab_v, src_v = rest[2 * NB:2 * NB + 3]
    sems = rest[:NB]
    sems_sc = rest[NB:2 * NB]

    cid = lax.axis_index("c")
    sid = lax.axis_index("s")
    wid = sid * NC + cid
    base_rows = wid * rows_per_tile

    off0 = sid * acc_mult * slc
    pltpu.sync_copy(zeros_hbm.at[pl.ds(0, half)], zbuf)
    for h in range(acc_mult * 2):
      pltpu.sync_copy(zbuf, acc.at[pl.ds(off0 + h * half, half)])

    if mode == 0:
      one16 = jnp.ones((L,), jnp.float32)

      @pl.loop(0, ROW // L)
      def _ones(i):
        ones_v[pl.ds(i * L, L)] = one16
    else:
      pltpu.sync_copy(tab_hbm, tab_v)

    plsc.subcore_barrier()

    def load_chunk(c, b):
      r = base_rows + c * K
      pltpu.async_copy(dst_hbm.at[pl.ds(r, K)], dst_v.at[b], sems[b])
      if mode != 0:
        pltpu.async_copy(src_hbm.at[pl.ds(r, K)], src_v.at[b], sems[b])

    def wait_chunk(c, b):
      r = base_rows + c * K
      pltpu.make_async_copy(dst_hbm.at[pl.ds(r, K)], dst_v.at[b], sems[b]).wait()
      if mode != 0:
        pltpu.make_async_copy(src_hbm.at[pl.ds(r, K)], src_v.at[b],
                              sems[b]).wait()

    def compute_chunk(b):
      for j in range(K):
        for g in range(ROW // L):
          col = g * L
          s16 = src_v[b, j, pl.ds(col, L)]
          vals = plsc.load_gather(tab_v, [s16])
          if mode == 2:
            d16 = dst_v[b, j, pl.ds(col, L)]
            route = jnp.where(vals < 0.0, jnp.int32(n_pad), jnp.int32(0))
            dst_v[b, j, pl.ds(col, L)] = d16 + route
          val_v[b, j, pl.ds(col, L)] = vals

    def issue_scatter(b):
      for j in range(K):
        src_ref = ones_v if mode == 0 else val_v.at[b, j]
        pltpu.async_copy(src_ref, acc.at[dst_v.at[b, j]], sems_sc[b], add=True)

    def drain_scatter(b):
      for j in range(K):
        src_ref = ones_v if mode == 0 else val_v.at[b, j]
        pltpu.make_async_copy(src_ref, acc.at[dst_v.at[b, j]],
                              sems_sc[b]).wait()

    load_chunk(0, 0)
    load_chunk(1, 1)

    @pl.loop(0, chunks // NB)
    def _main(oc):
      for t in range(NB):
        c = oc * NB + t
        bd = (t + 2) % NB
        if t >= 2:
          drain_scatter(bd)
        else:
          @pl.when(oc > 0)
          def _dr(bd=bd):
            drain_scatter(bd)
        if t < 2:
          load_chunk(c + 2, bd)
        else:
          @pl.when(oc < chunks // NB - 1)
          def _pf(c=c, bd=bd):
            load_chunk(c + 2, bd)
        wait_chunk(c, t)
        if mode != 0:
          compute_chunk(t)
        issue_scatter(t)

    drain_scatter(2)
    drain_scatter(3)

    plsc.subcore_barrier()

    for h in range(acc_mult * 2):
      pltpu.sync_copy(acc.at[pl.ds(off0 + h * half, half)], zbuf)
      pltpu.sync_copy(zbuf, out_hbm.at[pl.ds(cid * acc_n + off0 + h * half,
                                             half)])

  out_type = jax.ShapeDtypeStruct((NC * acc_n,), jnp.float32)
  return pl.kernel(
      body, out_type=out_type, mesh=mesh, scratch_types=scratch,
      compiler_params=pltpu.CompilerParams(needs_layout_passes=False))


def _tc1_body(degp_ref, x_ref, dis_ref, y_ref):
  deg = degp_ref[0] + degp_ref[1] + 1.0
  dis = lax.rsqrt(deg)
  dis_ref[...] = dis
  y_ref[...] = x_ref[...] * dis


def _tc2_body(t1p_ref, y_ref, dis_ref, sy_ref):
  t1 = t1p_ref[0] + t1p_ref[1]
  dis = dis_ref[...]
  sy_ref[...] = dis * dis * (t1 + y_ref[...])


def _tc3_body(tp_ref, sy_ref, dis_ref, par_ref, o0_ref, o1_ref):
  tu = tp_ref[0, 0] + tp_ref[1, 0]
  tv = tp_ref[0, 1] + tp_ref[1, 1]
  sy = sy_ref[...]
  dis = dis_ref[...]
  a = dis * (tu + jnp.maximum(sy, 0.0))
  c = dis * (tv + jnp.minimum(sy, 0.0))
  l0 = a * par_ref[0] + c * par_ref[2] + par_ref[4]
  l1 = a * par_ref[1] + c * par_ref[3] + par_ref[5]
  m = jnp.maximum(l0, l1)
  lse = m + jnp.log(jnp.exp(l0 - m) + jnp.exp(l1 - m))
  o0_ref[...] = l0 - lse
  o1_ref[...] = l1 - lse


def kernel(x, edge_index, W1, b1, W2, b2):
  n = x.shape[0]
  e = edge_index.shape[1]
  n_pad = ((n + 1 + 127) // 128) * 128
  r = n_pad // 128
  chunk_e = K * ROW
  chunks = -(-e // (NW * chunk_e))
  chunks = -(-chunks // NB) * NB
  e_pad = NW * chunks * chunk_e

  src = jnp.concatenate(
      [edge_index[0], jnp.zeros((e_pad - e,), jnp.int32)]).reshape(-1, ROW)
  dst = jnp.concatenate(
      [edge_index[1], jnp.full((e_pad - e,), n, jnp.int32)]).reshape(-1, ROW)
  xp = jnp.pad(x[:, 0], (0, n_pad - n)).reshape(r, 128)

  deg_pass = _make_edge_pass(0, n_pad, chunks)
  sum_pass = _make_edge_pass(1, n_pad, chunks)
  sign_pass = _make_edge_pass(2, n_pad, chunks)

  zeros1 = jnp.zeros((n_pad,), jnp.float32)
  zeros2 = jnp.zeros((2 * n_pad,), jnp.float32)

  degp = deg_pass(dst, zeros1)

  tc1 = pl.pallas_call(
      _tc1_body,
      out_shape=[jax.ShapeDtypeStruct((r, 128), jnp.float32)] * 2,
  )
  dis2, y2 = tc1(degp.reshape(NC, r, 128), xp)

  t1p = sum_pass(src, dst, y2.reshape(-1), zeros1)

  tc2 = pl.pallas_call(
      _tc2_body,
      out_shape=jax.ShapeDtypeStruct((r, 128), jnp.float32),
  )
  sy2 = tc2(t1p.reshape(NC, r, 128), y2, dis2)

  tp = sign_pass(src, dst, sy2.reshape(-1), zeros2)

  w1 = W1[0]
  wp2 = jnp.maximum(w1, 0.0) @ W2
  wm2 = jnp.minimum(w1, 0.0) @ W2
  par = jnp.stack([wp2[0], wp2[1], wm2[0], wm2[1], b2[0], b2[1],
                   jnp.float32(0), jnp.float32(0)])

  tc3 = pl.pallas_call(
      _tc3_body,
      in_specs=[
          pl.BlockSpec(),
          pl.BlockSpec(),
          pl.BlockSpec(),
          pl.BlockSpec(memory_space=pltpu.SMEM),
      ],
      out_shape=[jax.ShapeDtypeStruct((r, 128), jnp.float32)] * 2,
  )
  o0, o1 = tc3(tp.reshape(NC, 2, r, 128), sy2, dis2, par)

  return jnp.stack([o0.reshape(-1)[:n], o1.reshape(-1)[:n]], axis=1)


# speedup vs baseline: 350.7801x; 1.0271x over previous
import functools

import jax
import jax.numpy as jnp
from jax import lax
from jax.experimental import pallas as pl
from jax.experimental.pallas import tpu as pltpu
from jax.experimental.pallas import tpu_sc as plsc

NC = 2
NS = 16
NW = NC * NS
L = 16
ROW = 128
K = 8
NB = 4


def _make_edge_pass(mode, n_pad, chunks):
  acc_mult = 2 if mode == 2 else 1
  acc_n = n_pad * acc_mult
  slc = n_pad // NS
  rows_per_tile = chunks * K
  mesh = plsc.VectorSubcoreMesh(core_axis_name="c", subcore_axis_name="s")

  acc_dt = jnp.float32
  kr = K * ROW
  scratch = [
      pltpu.VMEM_SHARED((acc_n,), acc_dt),
      pltpu.VMEM((NB, K, ROW), jnp.int32),
  ]
  scratch += [pltpu.SemaphoreType.DMA] * NB
  scratch += [pltpu.SemaphoreType.DMA] * NB
  if mode == 0:
    scratch.append(pltpu.VMEM((ROW,), acc_dt))
    scratch.append(pltpu.VMEM((acc_mult * slc,), acc_dt))
  else:
    scratch.append(pltpu.VMEM((NB, kr), jnp.float32))
    scratch.append(pltpu.VMEM((n_pad,), jnp.float32))
    scratch.append(pltpu.VMEM((NB, K, ROW), jnp.int32))
  if mode == 2:
    scratch.append(pltpu.VMEM((NB, K, ROW), jnp.int32))

  def body(*refs):
    if mode == 0:
      (dst_hbm, zeros_hbm, out_hbm, acc, dst_v, *rest) = refs
      ones_v, stage = rest[2 * NB:2 * NB + 2]
    elif mode == 1:
      (src_hbm, dst_hbm, tab_hbm, zeros_hbm, out_hbm, acc, dst_v, *rest) = refs
      val_v, tab_v, src_v = rest[2 * NB:2 * NB + 3]
      stage = tab_v
    else:
      (src_hbm, dst_hbm, tab_hbm, zeros_hbm, out_hbm, acc, dst_v, *rest) = refs
      val_v, tab_v, src_v, idx_v = rest[2 * NB:2 * NB + 4]
      stage = tab_v
    sems = rest[:NB]
    sems_sc = rest[NB:2 * NB]

    cid = lax.axis_index("c")
    sid = lax.axis_index("s")
    wid = sid * NC + cid
    base_rows = wid * rows_per_tile
    ams = acc_mult * slc

    off0 = sid * ams
    pltpu.sync_copy(zeros_hbm.at[pl.ds(0, ams)], stage.at[pl.ds(0, ams)])
    pltpu.sync_copy(stage.at[pl.ds(0, ams)], acc.at[pl.ds(off0, ams)])

    if mode == 0:
      one16 = jnp.ones((L,), acc_dt)
      for i in range(ROW // L):
        ones_v[pl.ds(i * L, L)] = one16
    else:
      pltpu.sync_copy(tab_hbm, tab_v)

    plsc.subcore_barrier()

    def load_chunk(c, b):
      r = base_rows + c * K
      pltpu.async_copy(dst_hbm.at[pl.ds(r, K)], dst_v.at[b], sems[b])
      if mode != 0:
        pltpu.async_copy(src_hbm.at[pl.ds(r, K)], src_v.at[b], sems[b])

    def wait_chunk(c, b):
      r = base_rows + c * K
      pltpu.make_async_copy(dst_hbm.at[pl.ds(r, K)], dst_v.at[b], sems[b]).wait()
      if mode != 0:
        pltpu.make_async_copy(src_hbm.at[pl.ds(r, K)], src_v.at[b],
                              sems[b]).wait()

    def compute_chunk(b):
      for j in range(K):
        for g in range(ROW // L):
          col = g * L
          s16 = src_v[b, j, pl.ds(col, L)]
          vals = plsc.load_gather(tab_v, [s16])
          if mode == 2:
            d16 = dst_v[b, j, pl.ds(col, L)]
            route = jnp.where(vals < 0.0, jnp.int32(n_pad), jnp.int32(0))
            idx_v[b, j, pl.ds(col, L)] = d16 + route
          val_v[b, pl.ds(j * ROW + col, L)] = vals

    def issue_scatter(b):
      iv = dst_v if mode != 2 else idx_v
      for j in range(K):
        src_ref = ones_v if mode == 0 else val_v.at[b, pl.ds(j * ROW, ROW)]
        pltpu.async_copy(src_ref, acc.at[iv.at[b, j]], sems_sc[b], add=True)

    def drain_scatter(b):
      if mode == 0:
        pltpu.make_async_copy(dst_hbm.at[pl.ds(0, K)], dst_v.at[b],
                              sems_sc[b]).wait()
      else:
        pltpu.make_async_copy(zeros_hbm.at[pl.ds(0, kr)], val_v.at[b],
                              sems_sc[b]).wait()

    load_chunk(0, 0)
    load_chunk(1, 1)

    @pl.loop(0, chunks // NB)
    def _main(oc):
      for t in range(NB):
        c = oc * NB + t
        bd = (t + 2) % NB
        if t >= 2:
          drain_scatter(bd)
        else:
          @pl.when(oc > 0)
          def _dr(bd=bd):
            drain_scatter(bd)
        if t < 2:
          load_chunk(c + 2, bd)
        else:
          @pl.when(oc < chunks // NB - 1)
          def _pf(c=c, bd=bd):
            load_chunk(c + 2, bd)
        wait_chunk(c, t)
        if mode != 0:
          compute_chunk(t)
        issue_scatter(t)

    drain_scatter(2)
    drain_scatter(3)

    plsc.subcore_barrier()

    pltpu.sync_copy(acc.at[pl.ds(off0, ams)], stage.at[pl.ds(0, ams)])
    pltpu.sync_copy(stage.at[pl.ds(0, ams)],
                    out_hbm.at[pl.ds(cid * acc_n + off0, ams)])

  out_type = jax.ShapeDtypeStruct((NC * acc_n,), acc_dt)
  return pl.kernel(
      body, out_type=out_type, mesh=mesh, scratch_types=scratch,
      compiler_params=pltpu.CompilerParams(needs_layout_passes=False))


def _tc1_body(degp_ref, x_ref, dis_ref, y_ref):
  deg = degp_ref[0] + degp_ref[1] + 1.0
  dis = lax.rsqrt(deg)
  dis_ref[...] = dis
  y_ref[...] = x_ref[...] * dis


def _tc2_body(t1p_ref, y_ref, dis_ref, sy_ref):
  t1 = t1p_ref[0] + t1p_ref[1]
  dis = dis_ref[...]
  sy_ref[...] = dis * dis * (t1 + y_ref[...])


def _tc3_body(tp_ref, sy_ref, dis_ref, par_ref, o0_ref, o1_ref):
  tu = tp_ref[0, 0] + tp_ref[1, 0]
  tv = tp_ref[0, 1] + tp_ref[1, 1]
  sy = sy_ref[...]
  dis = dis_ref[...]
  a = dis * (tu + jnp.maximum(sy, 0.0))
  c = dis * (tv + jnp.minimum(sy, 0.0))
  l0 = a * par_ref[0] + c * par_ref[2] + par_ref[4]
  l1 = a * par_ref[1] + c * par_ref[3] + par_ref[5]
  m = jnp.maximum(l0, l1)
  lse = m + jnp.log(jnp.exp(l0 - m) + jnp.exp(l1 - m))
  o0_ref[...] = l0 - lse
  o1_ref[...] = l1 - lse


def kernel(x, edge_index, W1, b1, W2, b2):
  n = x.shape[0]
  e = edge_index.shape[1]
  n_pad = ((n + 1 + 127) // 128) * 128
  r = n_pad // 128
  chunk_e = K * ROW
  chunks = -(-e // (NW * chunk_e))
  chunks = -(-chunks // NB) * NB
  e_pad = NW * chunks * chunk_e

  src = jnp.concatenate(
      [edge_index[0], jnp.zeros((e_pad - e,), jnp.int32)]).reshape(-1, ROW)
  dst = jnp.concatenate(
      [edge_index[1], jnp.full((e_pad - e,), n, jnp.int32)]).reshape(-1, ROW)
  xp = jnp.pad(x[:, 0], (0, n_pad - n)).reshape(r, 128)

  deg_pass = _make_edge_pass(0, n_pad, chunks)
  sum_pass = _make_edge_pass(1, n_pad, chunks)
  sign_pass = _make_edge_pass(2, n_pad, chunks)

  zeros1 = jnp.zeros((n_pad,), jnp.float32)
  zeros2 = jnp.zeros((2 * n_pad,), jnp.float32)
  degp = deg_pass(dst, zeros1)

  tc1 = pl.pallas_call(
      _tc1_body,
      out_shape=[jax.ShapeDtypeStruct((r, 128), jnp.float32)] * 2,
  )
  dis2, y2 = tc1(degp.reshape(NC, r, 128), xp)

  t1p = sum_pass(src, dst, y2.reshape(-1), zeros1)

  tc2 = pl.pallas_call(
      _tc2_body,
      out_shape=jax.ShapeDtypeStruct((r, 128), jnp.float32),
  )
  sy2 = tc2(t1p.reshape(NC, r, 128), y2, dis2)

  tp = sign_pass(src, dst, sy2.reshape(-1), zeros2)

  w1 = W1[0]
  wp2 = jnp.maximum(w1, 0.0) @ W2
  wm2 = jnp.minimum(w1, 0.0) @ W2
  par = jnp.stack([wp2[0], wp2[1], wm2[0], wm2[1], b2[0], b2[1],
                   jnp.float32(0), jnp.float32(0)])

  tc3 = pl.pallas_call(
      _tc3_body,
      in_specs=[
          pl.BlockSpec(),
          pl.BlockSpec(),
          pl.BlockSpec(),
          pl.BlockSpec(memory_space=pltpu.SMEM),
      ],
      out_shape=[jax.ShapeDtypeStruct((r, 128), jnp.float32)] * 2,
  )
  o0, o1 = tc3(tp.reshape(NC, 2, r, 128), sy2, dis2, par)

  return jnp.stack([o0.reshape(-1)[:n], o1.reshape(-1)[:n]], axis=1)


# speedup vs baseline: 404.8727x; 1.1542x over previous
import functools

import jax
import jax.numpy as jnp
from jax import lax
from jax.experimental import pallas as pl
from jax.experimental.pallas import tpu as pltpu
from jax.experimental.pallas import tpu_sc as plsc

NC = 2
NS = 16
NW = NC * NS
L = 16
ROW = 128
K = 8
NB = 4


def _make_edge_pass(mode, n_pad, chunks):
  acc_mult = 2 if mode == 2 else 1
  acc_n = n_pad * acc_mult
  slc = n_pad // NS
  rows_per_tile = chunks * K
  mesh = plsc.VectorSubcoreMesh(core_axis_name="c", subcore_axis_name="s")

  acc_dt = jnp.float32
  kr = K * ROW
  scratch = [
      pltpu.VMEM_SHARED((acc_n,), acc_dt),
      pltpu.VMEM((NB, K, ROW), jnp.int32),
  ]
  scratch += [pltpu.SemaphoreType.DMA] * NB
  scratch += [pltpu.SemaphoreType.DMA] * NB
  if mode == 0:
    scratch.append(pltpu.VMEM((K, ROW), acc_dt))
    scratch.append(pltpu.VMEM((acc_mult * slc,), acc_dt))
  else:
    scratch.append(pltpu.VMEM((NB, K, ROW), jnp.float32))
    scratch.append(pltpu.VMEM((n_pad,), jnp.float32))
    scratch.append(pltpu.VMEM((NB, K, ROW), jnp.int32))
  if mode == 2:
    scratch.append(pltpu.VMEM((NB, K, ROW), jnp.int32))

  def body(*refs):
    if mode == 0:
      (dst_hbm, zeros_hbm, out_hbm, acc, dst_v, *rest) = refs
      ones_v, stage = rest[2 * NB:2 * NB + 2]
    elif mode == 1:
      (src_hbm, dst_hbm, tab_hbm, zeros_hbm, out_hbm, acc, dst_v, *rest) = refs
      val_v, tab_v, src_v = rest[2 * NB:2 * NB + 3]
      stage = tab_v
    else:
      (src_hbm, dst_hbm, tab_hbm, zeros_hbm, out_hbm, acc, dst_v, *rest) = refs
      val_v, tab_v, src_v, idx_v = rest[2 * NB:2 * NB + 4]
      stage = tab_v
    sems = rest[:NB]
    sems_sc = rest[NB:2 * NB]

    cid = lax.axis_index("c")
    sid = lax.axis_index("s")
    wid = sid * NC + cid
    base_rows = wid * rows_per_tile
    ams = acc_mult * slc

    off0 = sid * ams
    pltpu.sync_copy(zeros_hbm.at[pl.ds(0, ams)], stage.at[pl.ds(0, ams)])
    pltpu.sync_copy(stage.at[pl.ds(0, ams)], acc.at[pl.ds(off0, ams)])

    if mode == 0:
      one16 = jnp.ones((L,), acc_dt)
      for j in range(K):
        for i in range(ROW // L):
          ones_v[j, pl.ds(i * L, L)] = one16
    else:
      pltpu.sync_copy(tab_hbm, tab_v)

    plsc.subcore_barrier()

    def load_chunk(c, b):
      r = base_rows + c * K
      pltpu.async_copy(dst_hbm.at[pl.ds(r, K)], dst_v.at[b], sems[b])
      if mode != 0:
        pltpu.async_copy(src_hbm.at[pl.ds(r, K)], src_v.at[b], sems[b])

    def wait_chunk(c, b):
      r = base_rows + c * K
      pltpu.make_async_copy(dst_hbm.at[pl.ds(r, K)], dst_v.at[b], sems[b]).wait()
      if mode != 0:
        pltpu.make_async_copy(src_hbm.at[pl.ds(r, K)], src_v.at[b],
                              sems[b]).wait()

    def compute_chunk(b):
      iv = dst_v if mode != 2 else idx_v
      for j in range(K):
        if mode != 0:
          for q in range(ROW // (4 * L)):
            cols = [q * 4 * L + i * L for i in range(4)]
            s = [src_v[b, j, pl.ds(c, L)] for c in cols]
            v = [plsc.load_gather(tab_v, [si]) for si in s]
            if mode == 2:
              d = [dst_v[b, j, pl.ds(c, L)] for c in cols]
              rt = [jnp.where(vi < 0.0, jnp.int32(n_pad), jnp.int32(0))
                    for vi in v]
              for c, di, ri in zip(cols, d, rt):
                idx_v[b, j, pl.ds(c, L)] = di + ri
            for c, vi in zip(cols, v):
              val_v[b, j, pl.ds(c, L)] = vi
        src_ref = ones_v.at[j] if mode == 0 else val_v.at[b, j]
        pltpu.async_copy(src_ref, acc.at[iv.at[b, j]], sems_sc[b], add=True)

    def drain_scatter(b):
      if mode == 0:
        pltpu.make_async_copy(dst_hbm.at[pl.ds(0, K)], dst_v.at[b],
                              sems_sc[b]).wait()
      else:
        pltpu.make_async_copy(src_hbm.at[pl.ds(0, K)], src_v.at[b],
                              sems_sc[b]).wait()

    load_chunk(0, 0)
    load_chunk(1, 1)

    @pl.loop(0, chunks // NB)
    def _main(oc):
      for t in range(NB):
        c = oc * NB + t
        bd = (t + 2) % NB
        if t >= 2:
          drain_scatter(bd)
        else:
          @pl.when(oc > 0)
          def _dr(bd=bd):
            drain_scatter(bd)
        if t < 2:
          load_chunk(c + 2, bd)
        else:
          @pl.when(oc < chunks // NB - 1)
          def _pf(c=c, bd=bd):
            load_chunk(c + 2, bd)
        wait_chunk(c, t)
        compute_chunk(t)

    drain_scatter(2)
    drain_scatter(3)

    plsc.subcore_barrier()

    pltpu.sync_copy(acc.at[pl.ds(off0, ams)], stage.at[pl.ds(0, ams)])
    pltpu.sync_copy(stage.at[pl.ds(0, ams)],
                    out_hbm.at[pl.ds(cid * acc_n + off0, ams)])

  out_type = jax.ShapeDtypeStruct((NC * acc_n,), acc_dt)
  return pl.kernel(
      body, out_type=out_type, mesh=mesh, scratch_types=scratch,
      compiler_params=pltpu.CompilerParams(needs_layout_passes=False))


def _tc1_body(degp_ref, x_ref, dis_ref, y_ref):
  deg = degp_ref[0] + degp_ref[1] + 1.0
  dis = lax.rsqrt(deg)
  dis_ref[...] = dis
  y_ref[...] = x_ref[...] * dis


def _tc2_body(t1p_ref, y_ref, dis_ref, sy_ref):
  t1 = t1p_ref[0] + t1p_ref[1]
  dis = dis_ref[...]
  sy_ref[...] = dis * dis * (t1 + y_ref[...])


def _tc3_body(tp_ref, sy_ref, dis_ref, par_ref, o0_ref, o1_ref):
  tu = tp_ref[0, 0] + tp_ref[1, 0]
  tv = tp_ref[0, 1] + tp_ref[1, 1]
  sy = sy_ref[...]
  dis = dis_ref[...]
  a = dis * (tu + jnp.maximum(sy, 0.0))
  c = dis * (tv + jnp.minimum(sy, 0.0))
  l0 = a * par_ref[0] + c * par_ref[2] + par_ref[4]
  l1 = a * par_ref[1] + c * par_ref[3] + par_ref[5]
  m = jnp.maximum(l0, l1)
  lse = m + jnp.log(jnp.exp(l0 - m) + jnp.exp(l1 - m))
  o0_ref[...] = l0 - lse
  o1_ref[...] = l1 - lse


def kernel(x, edge_index, W1, b1, W2, b2):
  n = x.shape[0]
  e = edge_index.shape[1]
  n_pad = ((n + 1 + 127) // 128) * 128
  r = n_pad // 128
  chunk_e = K * ROW
  chunks = -(-e // (NW * chunk_e))
  chunks = -(-chunks // NB) * NB
  e_pad = NW * chunks * chunk_e

  src = jnp.concatenate(
      [edge_index[0], jnp.zeros((e_pad - e,), jnp.int32)]).reshape(-1, ROW)
  dst = jnp.concatenate(
      [edge_index[1], jnp.full((e_pad - e,), n, jnp.int32)]).reshape(-1, ROW)
  xp = jnp.pad(x[:, 0], (0, n_pad - n)).reshape(r, 128)

  deg_pass = _make_edge_pass(0, n_pad, chunks)
  sum_pass = _make_edge_pass(1, n_pad, chunks)
  sign_pass = _make_edge_pass(2, n_pad, chunks)

  zeros1 = jnp.zeros((n_pad,), jnp.float32)
  zeros2 = jnp.zeros((2 * n_pad,), jnp.float32)
  degp = deg_pass(dst, zeros1)

  tc1 = pl.pallas_call(
      _tc1_body,
      out_shape=[jax.ShapeDtypeStruct((r, 128), jnp.float32)] * 2,
  )
  dis2, y2 = tc1(degp.reshape(NC, r, 128), xp)

  t1p = sum_pass(src, dst, y2.reshape(-1), zeros1)

  tc2 = pl.pallas_call(
      _tc2_body,
      out_shape=jax.ShapeDtypeStruct((r, 128), jnp.float32),
  )
  sy2 = tc2(t1p.reshape(NC, r, 128), y2, dis2)

  tp = sign_pass(src, dst, sy2.reshape(-1), zeros2)

  w1 = W1[0]
  wp2 = jnp.maximum(w1, 0.0) @ W2
  wm2 = jnp.minimum(w1, 0.0) @ W2
  par = jnp.stack([wp2[0], wp2[1], wm2[0], wm2[1], b2[0], b2[1],
                   jnp.float32(0), jnp.float32(0)])

  tc3 = pl.pallas_call(
      _tc3_body,
      in_specs=[
          pl.BlockSpec(),
          pl.BlockSpec(),
          pl.BlockSpec(),
          pl.BlockSpec(memory_space=pltpu.SMEM),
      ],
      out_shape=[jax.ShapeDtypeStruct((r, 128), jnp.float32)] * 2,
  )
  o0, o1 = tc3(tp.reshape(NC, 2, r, 128), sy2, dis2, par)

  return jnp.stack([o0.reshape(-1)[:n], o1.reshape(-1)[:n]], axis=1)


# speedup vs baseline: 404.9026x; 1.0001x over previous
import functools

import jax
import jax.numpy as jnp
from jax import lax
from jax.experimental import pallas as pl
from jax.experimental.pallas import tpu as pltpu
from jax.experimental.pallas import tpu_sc as plsc

NC = 2
NS = 16
NW = NC * NS
L = 16
ROW = 128
K = 8
NB = 4


def _make_edge_pass(mode, n_pad, chunks):
  acc_mult = 2 if mode == 2 else 1
  acc_n = n_pad * acc_mult
  slc = n_pad // NS
  rows_per_tile = chunks * K
  mesh = plsc.VectorSubcoreMesh(core_axis_name="c", subcore_axis_name="s")

  acc_dt = jnp.float32
  scratch = [
      pltpu.VMEM_SHARED((acc_n,), acc_dt),
      pltpu.VMEM((NB, K, ROW), jnp.int32),
  ]
  scratch += [pltpu.SemaphoreType.DMA] * NB
  scratch += [pltpu.SemaphoreType.DMA] * NB
  if mode == 0:
    scratch.append(pltpu.VMEM((K, ROW), acc_dt))
    scratch.append(pltpu.VMEM((acc_mult * slc,), acc_dt))
  else:
    scratch.append(pltpu.VMEM((NB, K, ROW), jnp.float32))
    scratch.append(pltpu.VMEM((n_pad,), jnp.float32))
    scratch.append(pltpu.VMEM((NB, K, ROW), jnp.int32))
  if mode == 2:
    scratch.append(pltpu.VMEM((NB, K, ROW), jnp.int32))

  def body(*refs):
    if mode == 0:
      (dst_hbm, zeros_hbm, out_hbm, acc, dst_v, *rest) = refs
      ones_v, stage = rest[2 * NB:2 * NB + 2]
    elif mode == 1:
      (src_hbm, dst_hbm, tab_hbm, zeros_hbm, out_hbm, acc, dst_v, *rest) = refs
      val_v, tab_v, src_v = rest[2 * NB:2 * NB + 3]
      stage = tab_v
    else:
      (src_hbm, dst_hbm, tab_hbm, zeros_hbm, out_hbm, acc, dst_v, *rest) = refs
      val_v, tab_v, src_v, idx_v = rest[2 * NB:2 * NB + 4]
      stage = tab_v
    sems = rest[:NB]
    sems_sc = rest[NB:2 * NB]

    cid = lax.axis_index("c")
    sid = lax.axis_index("s")
    wid = sid * NC + cid
    base_rows = wid * rows_per_tile
    ams = acc_mult * slc

    off0 = sid * ams
    pltpu.sync_copy(zeros_hbm.at[pl.ds(0, ams)], stage.at[pl.ds(0, ams)])
    pltpu.sync_copy(stage.at[pl.ds(0, ams)], acc.at[pl.ds(off0, ams)])

    if mode == 0:
      one16 = jnp.ones((L,), acc_dt)
      for j in range(K):
        for i in range(ROW // L):
          ones_v[j, pl.ds(i * L, L)] = one16
    else:
      pltpu.sync_copy(tab_hbm, tab_v)

    plsc.subcore_barrier()

    def load_chunk(c, b):
      r = base_rows + c * K
      pltpu.async_copy(dst_hbm.at[pl.ds(r, K)], dst_v.at[b], sems[b])
      if mode != 0:
        pltpu.async_copy(src_hbm.at[pl.ds(r, K)], src_v.at[b], sems[b])

    def wait_chunk(c, b):
      r = base_rows + c * K
      pltpu.make_async_copy(dst_hbm.at[pl.ds(r, K)], dst_v.at[b], sems[b]).wait()
      if mode != 0:
        pltpu.make_async_copy(src_hbm.at[pl.ds(r, K)], src_v.at[b],
                              sems[b]).wait()

    def compute_chunk(b):
      iv = dst_v if mode != 2 else idx_v
      for j in range(K):
        if mode != 0:
          for q in range(ROW // (4 * L)):
            cols = [q * 4 * L + i * L for i in range(4)]
            s = [src_v[b, j, pl.ds(c, L)] for c in cols]
            v = [plsc.load_gather(tab_v, [si]) for si in s]
            if mode == 2:
              d = [dst_v[b, j, pl.ds(c, L)] for c in cols]
              rt = [jnp.where(vi < 0.0, jnp.int32(n_pad), jnp.int32(0))
                    for vi in v]
              for c, di, ri in zip(cols, d, rt):
                idx_v[b, j, pl.ds(c, L)] = di + ri
            for c, vi in zip(cols, v):
              val_v[b, j, pl.ds(c, L)] = vi
        src_ref = ones_v.at[j] if mode == 0 else val_v.at[b, j]
        pltpu.async_copy(src_ref, acc.at[iv.at[b, j]], sems_sc[b], add=True)

    def drain_scatter(b):
      if mode == 0:
        pltpu.make_async_copy(dst_hbm.at[pl.ds(0, K)], dst_v.at[b],
                              sems_sc[b]).wait()
      else:
        pltpu.make_async_copy(src_hbm.at[pl.ds(0, K)], src_v.at[b],
                              sems_sc[b]).wait()

    load_chunk(0, 0)
    load_chunk(1, 1)

    @pl.loop(0, chunks // NB)
    def _main(oc):
      for t in range(NB):
        c = oc * NB + t
        bd = (t + 2) % NB
        if t >= 2:
          drain_scatter(bd)
        else:
          @pl.when(oc > 0)
          def _dr(bd=bd):
            drain_scatter(bd)
        if t < 2:
          load_chunk(c + 2, bd)
        else:
          @pl.when(oc < chunks // NB - 1)
          def _pf(c=c, bd=bd):
            load_chunk(c + 2, bd)
        wait_chunk(c, t)
        compute_chunk(t)

    drain_scatter(2)
    drain_scatter(3)

    plsc.subcore_barrier()

    pltpu.sync_copy(acc.at[pl.ds(off0, ams)], stage.at[pl.ds(0, ams)])
    pltpu.sync_copy(stage.at[pl.ds(0, ams)],
                    out_hbm.at[pl.ds(cid * acc_n + off0, ams)])

  out_type = jax.ShapeDtypeStruct((NC * acc_n,), acc_dt)
  return pl.kernel(
      body, out_type=out_type, mesh=mesh, scratch_types=scratch,
      compiler_params=pltpu.CompilerParams(needs_layout_passes=False))


def _tc1_body(degp_ref, x_ref, dis_ref, y_ref):
  deg = degp_ref[0] + degp_ref[1] + 1.0
  dis = lax.rsqrt(deg)
  dis_ref[...] = dis
  y_ref[...] = x_ref[...] * dis


def _tc2_body(t1p_ref, y_ref, dis_ref, sy_ref):
  t1 = t1p_ref[0] + t1p_ref[1]
  dis = dis_ref[...]
  sy_ref[...] = dis * dis * (t1 + y_ref[...])


def _tc3_body(tp_ref, sy_ref, dis_ref, par_ref, o0_ref, o1_ref):
  tu = tp_ref[0, 0] + tp_ref[1, 0]
  tv = tp_ref[0, 1] + tp_ref[1, 1]
  sy = sy_ref[...]
  dis = dis_ref[...]
  a = dis * (tu + jnp.maximum(sy, 0.0))
  c = dis * (tv + jnp.minimum(sy, 0.0))
  l0 = a * par_ref[0] + c * par_ref[2] + par_ref[4]
  l1 = a * par_ref[1] + c * par_ref[3] + par_ref[5]
  m = jnp.maximum(l0, l1)
  lse = m + jnp.log(jnp.exp(l0 - m) + jnp.exp(l1 - m))
  o0_ref[...] = l0 - lse
  o1_ref[...] = l1 - lse


def kernel(x, edge_index, W1, b1, W2, b2):
  n = x.shape[0]
  e = edge_index.shape[1]
  n_pad = ((n + 1 + 127) // 128) * 128
  r = n_pad // 128
  chunk_e = K * ROW
  chunks = -(-e // (NW * chunk_e))
  chunks = -(-chunks // NB) * NB
  e_pad = NW * chunks * chunk_e

  src = jnp.concatenate(
      [edge_index[0], jnp.zeros((e_pad - e,), jnp.int32)]).reshape(-1, ROW)
  dst = jnp.concatenate(
      [edge_index[1], jnp.full((e_pad - e,), n, jnp.int32)]).reshape(-1, ROW)
  xp = jnp.pad(x[:, 0], (0, n_pad - n)).reshape(r, 128)

  deg_pass = _make_edge_pass(0, n_pad, chunks)
  sum_pass = _make_edge_pass(1, n_pad, chunks)
  sign_pass = _make_edge_pass(2, n_pad, chunks)

  zeros1 = jnp.zeros((n_pad,), jnp.float32)
  zeros2 = jnp.zeros((2 * n_pad,), jnp.float32)
  degp = deg_pass(dst, zeros1)

  tc1 = pl.pallas_call(
      _tc1_body,
      out_shape=[jax.ShapeDtypeStruct((r, 128), jnp.float32)] * 2,
  )
  dis2, y2 = tc1(degp.reshape(NC, r, 128), xp)

  t1p = sum_pass(src, dst, y2.reshape(-1), zeros1)

  tc2 = pl.pallas_call(
      _tc2_body,
      out_shape=jax.ShapeDtypeStruct((r, 128), jnp.float32),
  )
  sy2 = tc2(t1p.reshape(NC, r, 128), y2, dis2)

  tp = sign_pass(src, dst, sy2.reshape(-1), zeros2)

  w1 = W1[0]
  wp2 = jnp.maximum(w1, 0.0) @ W2
  wm2 = jnp.minimum(w1, 0.0) @ W2
  par = jnp.stack([wp2[0], wp2[1], wm2[0], wm2[1], b2[0], b2[1],
                   jnp.float32(0), jnp.float32(0)])

  tc3 = pl.pallas_call(
      _tc3_body,
      in_specs=[
          pl.BlockSpec(),
          pl.BlockSpec(),
          pl.BlockSpec(),
          pl.BlockSpec(memory_space=pltpu.SMEM),
      ],
      out_shape=[jax.ShapeDtypeStruct((r, 128), jnp.float32)] * 2,
  )
  o0, o1 = tc3(tp.reshape(NC, 2, r, 128), sy2, dis2, par)

  return jnp.stack([o0.reshape(-1)[:n], o1.reshape(-1)[:n]], axis=1)
